# baseline (speedup 1.0000x reference)
import functools

import jax
import jax.numpy as jnp
from jax import lax
from jax.experimental import pallas as pl
from jax.experimental.pallas import tpu as pltpu
from jax.experimental.pallas import tpu_sc as plsc

_B = 20
_N = 500
_E = 16000
_IN = 128
_HID = 128
_OUT = 10
_TAU = 1.0

_NP = 512
_Q = 4
_RQ = _NP // _Q
_UNITS = _B * _Q
_NW = 32
_UPW = -(-_UNITS // _NW)


_BPC = _B // 2
_LU = _BPC * _Q


def _build_adj(edge_index):
    mesh = plsc.VectorSubcoreMesh(core_axis_name="c", subcore_axis_name="s")

    @functools.partial(
        pl.kernel,
        out_type=jax.ShapeDtypeStruct((_B, _NP, _NP), jnp.float32),
        mesh=mesh,
        scratch_types=[
            pltpu.VMEM((_E,), jnp.int32),
            pltpu.VMEM((_E,), jnp.int32),
            pltpu.VMEM((_NP,), jnp.float32),
            pltpu.VMEM((_RQ, _NP), jnp.float32),
            pltpu.SemaphoreType.DMA,
            pltpu.SemaphoreType.DMA,
        ],
        compiler_params=pltpu.CompilerParams(needs_layout_passes=False),
    )
    def k(e_hbm, a_hbm, row_v, col_v, dinv_v, a_loc, sem_e, sem_w):
        cid = lax.axis_index("c")
        sid = lax.axis_index("s")
        ones16 = jnp.ones((16,), jnp.float32)
        zeros16 = jnp.zeros((16,), jnp.float32)
        iota16 = lax.iota(jnp.int32, 16)

        base = jnp.where(sid < 8, sid * 3, 24 + (sid - 8) * 2)
        nu = jnp.where(sid < 8, 3, 2)
        for u in range(3):

            @pl.when(u < nu)
            def _(u=u):
                l = base + u
                bl = l // _Q
                b = cid * _BPC + bl
                lo = (l % _Q) * _RQ
                hi = lo + _RQ
                ec = pltpu.async_copy(e_hbm.at[b, 1], col_v, sem_e)
                er = pltpu.async_copy(e_hbm.at[b, 0], row_v, sem_e)
                if u > 0:
                    pltpu.make_async_copy(
                        a_loc, a_hbm.at[b, pl.ds(lo, _RQ)], sem_w).wait()

                @plsc.parallel_loop(0, _RQ, 1, unroll=2)
                def _zr(i):
                    for j in range(_NP // 16):
                        a_loc[i, pl.ds(j * 16, 16)] = zeros16

                ec.wait()

                @plsc.parallel_loop(0, _NP // 128, 1)
                def _zdeg(i):
                    for j in range(8):
                        dinv_v[pl.ds((i * 8 + j) * 16, 16)] = ones16

                @plsc.parallel_loop(0, _E // 128, 1, unroll=2)
                def _cnt(i):
                    for j in range(8):
                        cc = col_v[pl.ds((i * 8 + j) * 16, 16)]
                        plsc.addupdate_scatter(dinv_v, [cc], ones16)

                def rs(i, c):
                    for j in range(4):
                        dd = dinv_v[pl.ds((i * 4 + j) * 16, 16)]
                        bits = plsc.bitcast(dd, jnp.int32)
                        y = plsc.bitcast(
                            jnp.int32(0x5F3759DF)
                            - lax.shift_right_logical(bits, 1), jnp.float32)
                        for _ in range(3):
                            y = y * (1.5 - 0.5 * dd * y * y)
                        dinv_v[pl.ds((i * 4 + j) * 16, 16)] = y
                    return c
                lax.fori_loop(0, _NP // 64, rs, 0)
                er.wait()

                @plsc.parallel_loop(0, _E // 64, 1, unroll=2)
                def _ed(i):
                    for j in range(4):
                        sl = pl.ds((i * 4 + j) * 16, 16)
                        r = row_v[sl]
                        cc = col_v[sl]
                        m = jnp.logical_and(cc >= lo, cc < hi)
                        dr = plsc.load_gather(dinv_v, [r])
                        dc = plsc.load_gather(dinv_v, [cc])
                        cl = jnp.where(m, cc - lo, 0)
                        plsc.addupdate_scatter(a_loc, [cl, r], dr * dc, mask=m)

                for j in range(_RQ // 16):
                    lr = j * 16 + iota16
                    n = lo + lr
                    m = n < _N
                    dn = dinv_v[pl.ds(lo + j * 16, 16)]
                    plsc.addupdate_scatter(a_loc, [lr, jnp.where(m, n, 0)],
                                           dn * dn, mask=m)

                pltpu.async_copy(a_loc, a_hbm.at[b, pl.ds(lo, _RQ)], sem_w)

        pltpu.make_async_copy(a_loc, a_hbm.at[0, pl.ds(0, _RQ)], sem_w).wait()

    return k(edge_index)


def _gcn_body(a_ref, x_ref, w1_ref, b1_ref, w2_ref, b2_ref, o_ref):
    av = a_ref[0]
    c11 = (((1,), (1,)), ((), ()))
    c10 = (((1,), (0,)), ((), ()))
    xw1 = lax.dot_general(x_ref[0], w1_ref[...], c11,
                          preferred_element_type=jnp.float32)
    avs = av[:, :_N]
    h1 = jnp.maximum(
        lax.dot_general(avs, xw1, c10, preferred_element_type=jnp.float32)
        + b1_ref[...], 0.0)
    csum = lax.dot_general(jnp.ones((1, _NP), jnp.float32), av, c10,
                           preferred_element_type=jnp.float32)
    ch1 = lax.dot_general(csum, h1, c10, preferred_element_type=jnp.float32)
    o_ref[0] = lax.dot_general(ch1, w2_ref[...], c11,
                               preferred_element_type=jnp.float32) \
        * (1.0 / _N) + b2_ref[...]


def _gcn(a, xg, W1, b1, W2, b2):
    out = pl.pallas_call(
        _gcn_body,
        grid=(_B,),
        in_specs=[
            pl.BlockSpec((1, _NP, _NP), lambda b: (b, 0, 0)),
            pl.BlockSpec((1, _N, _IN), lambda b: (b, 0, 0)),
            pl.BlockSpec((_HID, _IN), lambda b: (0, 0)),
            pl.BlockSpec((1, _HID), lambda b: (0, 0)),
            pl.BlockSpec((_HID, _HID), lambda b: (0, 0)),
            pl.BlockSpec((1, _HID), lambda b: (0, 0)),
        ],
        out_specs=pl.BlockSpec((1, 1, _HID), lambda b: (b, 0, 0)),
        out_shape=jax.ShapeDtypeStruct((_B, 1, _HID), jnp.float32),
    )(a, xg, W1, b1.reshape(1, -1), W2, b2.reshape(1, -1))
    return out.reshape(_B, _HID)


def _head_body(x_ref, emb_ref, labr_ref, labc_ref, we_ref, be_ref,
               w1_ref, b1_ref, w2_ref, b2_ref, w3_ref, b3_ref,
               wd_ref, bd_ref, comb_ref, ximp_ref, loss_ref):
    c11 = (((1,), (1,)), ((), ()))
    f32 = jnp.float32
    x = x_ref[...]
    z = lax.dot_general(x, we_ref[...], c11, preferred_element_type=f32) \
        + be_ref[...]
    emb = emb_ref[...]
    nrm = jnp.sqrt(jnp.sum(emb * emb, axis=1, keepdims=True))
    node = emb / jnp.maximum(nrm, 1e-12)
    h = jnp.maximum(
        lax.dot_general(node, w1_ref[...], c11, preferred_element_type=f32)
        + b1_ref[...], 0.0)
    h = jnp.maximum(
        lax.dot_general(h, w2_ref[...], c11, preferred_element_type=f32)
        + b2_ref[...], 0.0)
    ximp_ref[...] = lax.dot_general(h, w3_ref[...], c11,
                                    preferred_element_type=f32) + b3_ref[...]
    comb = jnp.concatenate([z, node], axis=1)
    comb_ref[...] = comb
    logits = lax.dot_general(jnp.maximum(z, 0.0), wd_ref[...], c11,
                             preferred_element_type=f32) + bd_ref[...]
    gram = lax.dot_general(comb, comb, c11, preferred_element_type=f32)
    dist = -jnp.exp(gram * (1.0 / _TAU))
    labr = labr_ref[...]
    msk = (labr == labc_ref[...]).astype(f32)
    contrastive = jnp.sum((2.0 * msk - 1.0) * dist) / _B
    mx = jnp.max(logits, axis=1, keepdims=True)
    lse = jnp.log(jnp.sum(jnp.exp(logits - mx), axis=1, keepdims=True)) + mx
    logp = logits - lse
    oh = lax.broadcasted_iota(jnp.int32, (_B, _OUT), 1) == labr
    ce = -jnp.sum(jnp.where(oh, logp, 0.0)) / _B
    loss_ref[...] = (contrastive + ce).reshape(1, 1)


def _head(x, embs, labels, W_e, b_e, W_i1, b_i1, W_i2, b_i2, W_i3, b_i3,
          W_d, b_d):
    return pl.pallas_call(
        _head_body,
        out_shape=[
            jax.ShapeDtypeStruct((_B, 2 * _HID), jnp.float32),
            jax.ShapeDtypeStruct((_B, _IN), jnp.float32),
            jax.ShapeDtypeStruct((1, 1), jnp.float32),
        ],
    )(x, embs, labels.reshape(_B, 1), labels.reshape(1, _B),
      W_e, b_e.reshape(1, -1), W_i1, b_i1.reshape(1, -1),
      W_i2, b_i2.reshape(1, -1), W_i3, b_i3.reshape(1, -1),
      W_d, b_d.reshape(1, -1))


def kernel(x, labels, loc, subgraph_x, subgraph_edge_index, W_e, b_e,
           W_g1, b_g1, W_g2, b_g2, W_i1, b_i1, W_i2, b_i2, W_i3, b_i3,
           W_d, b_d):
    a = _build_adj(subgraph_edge_index)
    embs = _gcn(a, subgraph_x, W_g1, b_g1, W_g2, b_g2)
    combined, x_imp, loss = _head(x, embs, labels, W_e, b_e,
                                  W_i1, b_i1, W_i2, b_i2, W_i3, b_i3,
                                  W_d, b_d)
    return combined, x_imp, loss[0, 0]


# speedup vs baseline: 1.0073x; 1.0073x over previous
import functools

import jax
import jax.numpy as jnp
from jax import lax
from jax.experimental import pallas as pl
from jax.experimental.pallas import tpu as pltpu
from jax.experimental.pallas import tpu_sc as plsc

_B = 20
_N = 500
_E = 16000
_IN = 128
_HID = 128
_OUT = 10
_TAU = 1.0

_NP = 512
_Q = 4
_RQ = _NP // _Q
_UNITS = _B * _Q
_NW = 32
_UPW = -(-_UNITS // _NW)


_BPC = _B // 2
_LU = _BPC * _Q


def _build_adj(edge_index):
    mesh = plsc.VectorSubcoreMesh(core_axis_name="c", subcore_axis_name="s")

    @functools.partial(
        pl.kernel,
        out_type=jax.ShapeDtypeStruct((_B, _NP, _NP), jnp.float32),
        mesh=mesh,
        scratch_types=[
            pltpu.VMEM((_E,), jnp.int32),
            pltpu.VMEM((_E,), jnp.int32),
            pltpu.VMEM((_NP,), jnp.float32),
            pltpu.VMEM((_RQ, _NP), jnp.float32),
            pltpu.SemaphoreType.DMA,
            pltpu.SemaphoreType.DMA,
        ],
        compiler_params=pltpu.CompilerParams(needs_layout_passes=False),
    )
    def k(e_hbm, a_hbm, row_v, col_v, dinv_v, a_loc, sem_e, sem_w):
        cid = lax.axis_index("c")
        sid = lax.axis_index("s")
        ones16 = jnp.ones((16,), jnp.float32)
        zeros16 = jnp.zeros((16,), jnp.float32)
        iota16 = lax.iota(jnp.int32, 16)

        base = jnp.where(sid < 8, sid * 3, 24 + (sid - 8) * 2)
        nu = jnp.where(sid < 8, 3, 2)
        for u in range(3):

            @pl.when(u < nu)
            def _(u=u):
                l = base + u
                bl = l // _Q
                b = cid * _BPC + bl
                lo = (l % _Q) * _RQ
                hi = lo + _RQ
                ec = pltpu.async_copy(e_hbm.at[b, 1], col_v, sem_e)
                er = pltpu.async_copy(e_hbm.at[b, 0], row_v, sem_e)
                if u > 0:
                    pltpu.make_async_copy(
                        a_loc, a_hbm.at[b, pl.ds(lo, _RQ)], sem_w).wait()

                @plsc.parallel_loop(0, _RQ, 1, unroll=2)
                def _zr(i):
                    for j in range(_NP // 16):
                        a_loc[i, pl.ds(j * 16, 16)] = zeros16

                ec.wait()

                @plsc.parallel_loop(0, _NP // 128, 1)
                def _zdeg(i):
                    for j in range(8):
                        dinv_v[pl.ds((i * 8 + j) * 16, 16)] = ones16

                @plsc.parallel_loop(0, _E // 128, 1, unroll=2)
                def _cnt(i):
                    for j in range(8):
                        cc = col_v[pl.ds((i * 8 + j) * 16, 16)]
                        plsc.addupdate_scatter(dinv_v, [cc], ones16)

                def rs(i, c):
                    for j in range(4):
                        dd = dinv_v[pl.ds((i * 4 + j) * 16, 16)]
                        bits = plsc.bitcast(dd, jnp.int32)
                        y = plsc.bitcast(
                            jnp.int32(0x5F3759DF)
                            - lax.shift_right_logical(bits, 1), jnp.float32)
                        for _ in range(3):
                            y = y * (1.5 - 0.5 * dd * y * y)
                        dinv_v[pl.ds((i * 4 + j) * 16, 16)] = y
                    return c
                lax.fori_loop(0, _NP // 64, rs, 0)
                er.wait()

                @plsc.parallel_loop(0, _E // 64, 1, unroll=4)
                def _ed(i):
                    for j in range(4):
                        sl = pl.ds((i * 4 + j) * 16, 16)
                        r = row_v[sl]
                        cc = col_v[sl]
                        m = jnp.logical_and(cc >= lo, cc < hi)
                        dr = plsc.load_gather(dinv_v, [r])
                        dc = plsc.load_gather(dinv_v, [cc])
                        cl = jnp.where(m, cc - lo, 0)
                        plsc.addupdate_scatter(a_loc, [cl, r], dr * dc, mask=m)

                for j in range(_RQ // 16):
                    lr = j * 16 + iota16
                    n = lo + lr
                    m = n < _N
                    dn = dinv_v[pl.ds(lo + j * 16, 16)]
                    plsc.addupdate_scatter(a_loc, [lr, jnp.where(m, n, 0)],
                                           dn * dn, mask=m)

                pltpu.async_copy(a_loc, a_hbm.at[b, pl.ds(lo, _RQ)], sem_w)

        pltpu.make_async_copy(a_loc, a_hbm.at[0, pl.ds(0, _RQ)], sem_w).wait()

    return k(edge_index)


def _xw1_body(x_ref, w1_ref, o_ref):
    c11 = (((1,), (1,)), ((), ()))
    r = lax.dot_general(x_ref[0], w1_ref[...], c11,
                        preferred_element_type=jnp.float32)
    o_ref[0, :_N] = r
    o_ref[0, _N:] = jnp.zeros((_NP - _N, _HID), jnp.float32)


def _xw1(xg, W1):
    return pl.pallas_call(
        _xw1_body,
        grid=(_B,),
        in_specs=[
            pl.BlockSpec((1, _N, _IN), lambda b: (b, 0, 0)),
            pl.BlockSpec((_HID, _IN), lambda b: (0, 0)),
        ],
        out_specs=pl.BlockSpec((1, _NP, _HID), lambda b: (b, 0, 0)),
        out_shape=jax.ShapeDtypeStruct((_B, _NP, _HID), jnp.float32),
    )(xg, W1)


def _gcn_body(a_ref, xw_ref, b1_ref, w2_ref, b2_ref, o_ref):
    av = a_ref[0]
    c11 = (((1,), (1,)), ((), ()))
    c10 = (((1,), (0,)), ((), ()))
    h1 = jnp.maximum(
        lax.dot_general(av, xw_ref[0], c10, preferred_element_type=jnp.float32)
        + b1_ref[...], 0.0)
    csum = lax.dot_general(jnp.ones((1, _NP), jnp.float32), av, c10,
                           preferred_element_type=jnp.float32)
    ch1 = lax.dot_general(csum, h1, c10, preferred_element_type=jnp.float32)
    o_ref[0] = lax.dot_general(ch1, w2_ref[...], c11,
                               preferred_element_type=jnp.float32) \
        * (1.0 / _N) + b2_ref[...]


def _gcn(a, xw, b1, W2, b2):
    out = pl.pallas_call(
        _gcn_body,
        grid=(_B,),
        in_specs=[
            pl.BlockSpec((1, _NP, _NP), lambda b: (b, 0, 0)),
            pl.BlockSpec((1, _NP, _HID), lambda b: (b, 0, 0)),
            pl.BlockSpec((1, _HID), lambda b: (0, 0)),
            pl.BlockSpec((_HID, _HID), lambda b: (0, 0)),
            pl.BlockSpec((1, _HID), lambda b: (0, 0)),
        ],
        out_specs=pl.BlockSpec((1, 1, _HID), lambda b: (b, 0, 0)),
        out_shape=jax.ShapeDtypeStruct((_B, 1, _HID), jnp.float32),
    )(a, xw, b1.reshape(1, -1), W2, b2.reshape(1, -1))
    return out.reshape(_B, _HID)


def _head_body(x_ref, emb_ref, labr_ref, labc_ref, we_ref, be_ref,
               w1_ref, b1_ref, w2_ref, b2_ref, w3_ref, b3_ref,
               wd_ref, bd_ref, comb_ref, ximp_ref, loss_ref):
    c11 = (((1,), (1,)), ((), ()))
    f32 = jnp.float32
    x = x_ref[...]
    z = lax.dot_general(x, we_ref[...], c11, preferred_element_type=f32) \
        + be_ref[...]
    emb = emb_ref[...]
    nrm = jnp.sqrt(jnp.sum(emb * emb, axis=1, keepdims=True))
    node = emb / jnp.maximum(nrm, 1e-12)
    h = jnp.maximum(
        lax.dot_general(node, w1_ref[...], c11, preferred_element_type=f32)
        + b1_ref[...], 0.0)
    h = jnp.maximum(
        lax.dot_general(h, w2_ref[...], c11, preferred_element_type=f32)
        + b2_ref[...], 0.0)
    ximp_ref[...] = lax.dot_general(h, w3_ref[...], c11,
                                    preferred_element_type=f32) + b3_ref[...]
    comb = jnp.concatenate([z, node], axis=1)
    comb_ref[...] = comb
    logits = lax.dot_general(jnp.maximum(z, 0.0), wd_ref[...], c11,
                             preferred_element_type=f32) + bd_ref[...]
    gram = lax.dot_general(comb, comb, c11, preferred_element_type=f32)
    dist = -jnp.exp(gram * (1.0 / _TAU))
    labr = labr_ref[...]
    msk = (labr == labc_ref[...]).astype(f32)
    contrastive = jnp.sum((2.0 * msk - 1.0) * dist) / _B
    mx = jnp.max(logits, axis=1, keepdims=True)
    lse = jnp.log(jnp.sum(jnp.exp(logits - mx), axis=1, keepdims=True)) + mx
    logp = logits - lse
    oh = lax.broadcasted_iota(jnp.int32, (_B, _OUT), 1) == labr
    ce = -jnp.sum(jnp.where(oh, logp, 0.0)) / _B
    loss_ref[...] = (contrastive + ce).reshape(1, 1)


def _head(x, embs, labels, W_e, b_e, W_i1, b_i1, W_i2, b_i2, W_i3, b_i3,
          W_d, b_d):
    return pl.pallas_call(
        _head_body,
        out_shape=[
            jax.ShapeDtypeStruct((_B, 2 * _HID), jnp.float32),
            jax.ShapeDtypeStruct((_B, _IN), jnp.float32),
            jax.ShapeDtypeStruct((1, 1), jnp.float32),
        ],
    )(x, embs, labels.reshape(_B, 1), labels.reshape(1, _B),
      W_e, b_e.reshape(1, -1), W_i1, b_i1.reshape(1, -1),
      W_i2, b_i2.reshape(1, -1), W_i3, b_i3.reshape(1, -1),
      W_d, b_d.reshape(1, -1))


def kernel(x, labels, loc, subgraph_x, subgraph_edge_index, W_e, b_e,
           W_g1, b_g1, W_g2, b_g2, W_i1, b_i1, W_i2, b_i2, W_i3, b_i3,
           W_d, b_d):
    xw = _xw1(subgraph_x, W_g1)
    a = _build_adj(subgraph_edge_index)
    embs = _gcn(a, xw, b_g1, W_g2, b_g2)
    combined, x_imp, loss = _head(x, embs, labels, W_e, b_e,
                                  W_i1, b_i1, W_i2, b_i2, W_i3, b_i3,
                                  W_d, b_d)
    return combined, x_imp, loss[0, 0]


# speedup vs baseline: 1.0823x; 1.0745x over previous
import functools

import jax
import jax.numpy as jnp
from jax import lax
from jax.experimental import pallas as pl
from jax.experimental.pallas import tpu as pltpu
from jax.experimental.pallas import tpu_sc as plsc

_B = 20
_N = 500
_E = 16000
_IN = 128
_HID = 128
_OUT = 10
_TAU = 1.0

_NP = 512
_Q = 4
_RQ = _NP // _Q
_UNITS = _B * _Q
_NW = 32
_UPW = -(-_UNITS // _NW)


_BPC = _B // 2
_LU = _BPC * _Q


def _build_adj(edge_index):
    mesh = plsc.VectorSubcoreMesh(core_axis_name="c", subcore_axis_name="s")

    @functools.partial(
        pl.kernel,
        out_type=jax.ShapeDtypeStruct((_B, _NP, _NP), jnp.float32),
        mesh=mesh,
        scratch_types=[
            pltpu.VMEM((_E,), jnp.int32),
            pltpu.VMEM((_E,), jnp.int32),
            pltpu.VMEM((_NP,), jnp.float32),
            pltpu.VMEM((_RQ, _NP), jnp.float32),
            pltpu.SemaphoreType.DMA,
            pltpu.SemaphoreType.DMA,
        ],
        compiler_params=pltpu.CompilerParams(needs_layout_passes=False),
    )
    def k(e_hbm, a_hbm, row_v, col_v, dinv_v, a_loc, sem_e, sem_w):
        cid = lax.axis_index("c")
        sid = lax.axis_index("s")
        ones16 = jnp.ones((16,), jnp.float32)
        zeros16 = jnp.zeros((16,), jnp.float32)
        iota16 = lax.iota(jnp.int32, 16)

        base = jnp.where(sid < 8, sid * 3, 24 + (sid - 8) * 2)
        nu = jnp.where(sid < 8, 3, 2)
        for u in range(3):

            @pl.when(u < nu)
            def _(u=u):
                l = base + u
                bl = l // _Q
                b = cid * _BPC + bl
                lo = (l % _Q) * _RQ
                hi = lo + _RQ
                ec = pltpu.async_copy(e_hbm.at[b, 1], col_v, sem_e)
                er = pltpu.async_copy(e_hbm.at[b, 0], row_v, sem_e)
                if u > 0:
                    pltpu.make_async_copy(
                        a_loc, a_hbm.at[b, pl.ds(lo, _RQ)], sem_w).wait()

                @plsc.parallel_loop(0, _RQ, 1, unroll=2)
                def _zr(i):
                    for j in range(_NP // 16):
                        a_loc[i, pl.ds(j * 16, 16)] = zeros16

                ec.wait()

                @plsc.parallel_loop(0, _NP // 128, 1)
                def _zdeg(i):
                    for j in range(8):
                        dinv_v[pl.ds((i * 8 + j) * 16, 16)] = ones16

                @plsc.parallel_loop(0, _E // 128, 1, unroll=2)
                def _cnt(i):
                    for j in range(8):
                        cc = col_v[pl.ds((i * 8 + j) * 16, 16)]
                        plsc.addupdate_scatter(dinv_v, [cc], ones16)

                def rs(i, c):
                    for j in range(4):
                        dd = dinv_v[pl.ds((i * 4 + j) * 16, 16)]
                        bits = plsc.bitcast(dd, jnp.int32)
                        y = plsc.bitcast(
                            jnp.int32(0x5F3759DF)
                            - lax.shift_right_logical(bits, 1), jnp.float32)
                        for _ in range(3):
                            y = y * (1.5 - 0.5 * dd * y * y)
                        dinv_v[pl.ds((i * 4 + j) * 16, 16)] = y
                    return c
                lax.fori_loop(0, _NP // 64, rs, 0)
                er.wait()

                @plsc.parallel_loop(0, _E // 64, 1, unroll=2)
                def _ed(i):
                    for j in range(4):
                        sl = pl.ds((i * 4 + j) * 16, 16)
                        r = row_v[sl]
                        cc = col_v[sl]
                        m = jnp.logical_and(cc >= lo, cc < hi)
                        dr = plsc.load_gather(dinv_v, [r])
                        dc = plsc.load_gather(dinv_v, [cc])
                        cl = jnp.where(m, cc - lo, 0)
                        plsc.addupdate_scatter(a_loc, [cl, r], dr * dc, mask=m)

                for j in range(_RQ // 16):
                    lr = j * 16 + iota16
                    n = lo + lr
                    m = n < _N
                    dn = dinv_v[pl.ds(lo + j * 16, 16)]
                    plsc.addupdate_scatter(a_loc, [lr, jnp.where(m, n, 0)],
                                           dn * dn, mask=m)

                pltpu.async_copy(a_loc, a_hbm.at[b, pl.ds(lo, _RQ)], sem_w)

        pltpu.make_async_copy(a_loc, a_hbm.at[0, pl.ds(0, _RQ)], sem_w).wait()

    return k(edge_index)


def _xw1_body(x_ref, w1_ref, o_ref):
    c11 = (((1,), (1,)), ((), ()))
    r = lax.dot_general(x_ref[0], w1_ref[...], c11,
                        preferred_element_type=jnp.float32)
    o_ref[0, :_N] = r
    o_ref[0, _N:] = jnp.zeros((_NP - _N, _HID), jnp.float32)


def _xw1(xg, W1):
    return pl.pallas_call(
        _xw1_body,
        grid=(_B,),
        in_specs=[
            pl.BlockSpec((1, _N, _IN), lambda b: (b, 0, 0)),
            pl.BlockSpec((_HID, _IN), lambda b: (0, 0)),
        ],
        out_specs=pl.BlockSpec((1, _NP, _HID), lambda b: (b, 0, 0)),
        out_shape=jax.ShapeDtypeStruct((_B, _NP, _HID), jnp.float32),
    )(xg, W1)


_GB = 2


def _gcn_body(a_ref, xw_ref, b1_ref, w2_ref, b2_ref, o_ref):
    c11 = (((1,), (1,)), ((), ()))
    c10 = (((1,), (0,)), ((), ()))
    for t in range(_GB):
        av = a_ref[t]
        h1 = jnp.maximum(
            lax.dot_general(av, xw_ref[t], c10,
                            preferred_element_type=jnp.float32)
            + b1_ref[...], 0.0)
        csum = lax.dot_general(jnp.ones((1, _NP), jnp.float32), av, c10,
                               preferred_element_type=jnp.float32)
        ch1 = lax.dot_general(csum, h1, c10,
                              preferred_element_type=jnp.float32)
        o_ref[t] = lax.dot_general(ch1, w2_ref[...], c11,
                                   preferred_element_type=jnp.float32) \
            * (1.0 / _N) + b2_ref[...]


def _gcn(a, xw, b1, W2, b2):
    out = pl.pallas_call(
        _gcn_body,
        grid=(_B // _GB,),
        in_specs=[
            pl.BlockSpec((_GB, _NP, _NP), lambda b: (b, 0, 0)),
            pl.BlockSpec((_GB, _NP, _HID), lambda b: (b, 0, 0)),
            pl.BlockSpec((1, _HID), lambda b: (0, 0)),
            pl.BlockSpec((_HID, _HID), lambda b: (0, 0)),
            pl.BlockSpec((1, _HID), lambda b: (0, 0)),
        ],
        out_specs=pl.BlockSpec((_GB, 1, _HID), lambda b: (b, 0, 0)),
        out_shape=jax.ShapeDtypeStruct((_B, 1, _HID), jnp.float32),
    )(a, xw, b1.reshape(1, -1), W2, b2.reshape(1, -1))
    return out.reshape(_B, _HID)


def _head_body(x_ref, emb_ref, labr_ref, labc_ref, we_ref, be_ref,
               w1_ref, b1_ref, w2_ref, b2_ref, w3_ref, b3_ref,
               wd_ref, bd_ref, comb_ref, ximp_ref, loss_ref):
    c11 = (((1,), (1,)), ((), ()))
    f32 = jnp.float32
    x = x_ref[...]
    z = lax.dot_general(x, we_ref[...], c11, preferred_element_type=f32) \
        + be_ref[...]
    emb = emb_ref[...]
    nrm = jnp.sqrt(jnp.sum(emb * emb, axis=1, keepdims=True))
    node = emb / jnp.maximum(nrm, 1e-12)
    h = jnp.maximum(
        lax.dot_general(node, w1_ref[...], c11, preferred_element_type=f32)
        + b1_ref[...], 0.0)
    h = jnp.maximum(
        lax.dot_general(h, w2_ref[...], c11, preferred_element_type=f32)
        + b2_ref[...], 0.0)
    ximp_ref[...] = lax.dot_general(h, w3_ref[...], c11,
                                    preferred_element_type=f32) + b3_ref[...]
    comb = jnp.concatenate([z, node], axis=1)
    comb_ref[...] = comb
    logits = lax.dot_general(jnp.maximum(z, 0.0), wd_ref[...], c11,
                             preferred_element_type=f32) + bd_ref[...]
    gram = lax.dot_general(comb, comb, c11, preferred_element_type=f32)
    dist = -jnp.exp(gram * (1.0 / _TAU))
    labr = labr_ref[...]
    msk = (labr == labc_ref[...]).astype(f32)
    contrastive = jnp.sum((2.0 * msk - 1.0) * dist) / _B
    mx = jnp.max(logits, axis=1, keepdims=True)
    lse = jnp.log(jnp.sum(jnp.exp(logits - mx), axis=1, keepdims=True)) + mx
    logp = logits - lse
    oh = lax.broadcasted_iota(jnp.int32, (_B, _OUT), 1) == labr
    ce = -jnp.sum(jnp.where(oh, logp, 0.0)) / _B
    loss_ref[...] = (contrastive + ce).reshape(1, 1)


def _head(x, embs, labels, W_e, b_e, W_i1, b_i1, W_i2, b_i2, W_i3, b_i3,
          W_d, b_d):
    return pl.pallas_call(
        _head_body,
        out_shape=[
            jax.ShapeDtypeStruct((_B, 2 * _HID), jnp.float32),
            jax.ShapeDtypeStruct((_B, _IN), jnp.float32),
            jax.ShapeDtypeStruct((1, 1), jnp.float32),
        ],
    )(x, embs, labels.reshape(_B, 1), labels.reshape(1, _B),
      W_e, b_e.reshape(1, -1), W_i1, b_i1.reshape(1, -1),
      W_i2, b_i2.reshape(1, -1), W_i3, b_i3.reshape(1, -1),
      W_d, b_d.reshape(1, -1))


def kernel(x, labels, loc, subgraph_x, subgraph_edge_index, W_e, b_e,
           W_g1, b_g1, W_g2, b_g2, W_i1, b_i1, W_i2, b_i2, W_i3, b_i3,
           W_d, b_d):
    xw = _xw1(subgraph_x, W_g1)
    a = _build_adj(subgraph_edge_index)
    embs = _gcn(a, xw, b_g1, W_g2, b_g2)
    combined, x_imp, loss = _head(x, embs, labels, W_e, b_e,
                                  W_i1, b_i1, W_i2, b_i2, W_i3, b_i3,
                                  W_d, b_d)
    return combined, x_imp, loss[0, 0]


# speedup vs baseline: 1.1059x; 1.0218x over previous
import functools

import jax
import jax.numpy as jnp
from jax import lax
from jax.experimental import pallas as pl
from jax.experimental.pallas import tpu as pltpu
from jax.experimental.pallas import tpu_sc as plsc

_B = 20
_N = 500
_E = 16000
_IN = 128
_HID = 128
_OUT = 10
_TAU = 1.0

_NP = 512
_Q = 4
_RQ = _NP // _Q
_UNITS = _B * _Q
_NW = 32
_UPW = -(-_UNITS // _NW)


_BPC = _B // 2
_LU = _BPC * _Q


def _build_adj(edge_index):
    mesh = plsc.VectorSubcoreMesh(core_axis_name="c", subcore_axis_name="s")

    @functools.partial(
        pl.kernel,
        out_type=jax.ShapeDtypeStruct((_B, _NP, _NP), jnp.float32),
        mesh=mesh,
        scratch_types=[
            pltpu.VMEM((_E,), jnp.int32),
            pltpu.VMEM((_E,), jnp.int32),
            pltpu.VMEM((_NP,), jnp.float32),
            pltpu.VMEM((_RQ, _NP), jnp.float32),
            pltpu.SemaphoreType.DMA,
            pltpu.SemaphoreType.DMA,
        ],
        compiler_params=pltpu.CompilerParams(needs_layout_passes=False),
    )
    def k(e_hbm, a_hbm, row_v, col_v, dinv_v, a_loc, sem_e, sem_w):
        cid = lax.axis_index("c")
        sid = lax.axis_index("s")
        ones16 = jnp.ones((16,), jnp.float32)
        zeros16 = jnp.zeros((16,), jnp.float32)
        iota16 = lax.iota(jnp.int32, 16)

        base = jnp.where(sid < 8, sid * 3, 24 + (sid - 8) * 2)
        nu = jnp.where(sid < 8, 3, 2)
        for u in range(3):

            @pl.when(u < nu)
            def _(u=u):
                l = base + u
                bl = l // _Q
                b = cid * _BPC + bl
                lo = (l % _Q) * _RQ
                hi = lo + _RQ
                ec = pltpu.async_copy(e_hbm.at[b, 1], col_v, sem_e)
                er = pltpu.async_copy(e_hbm.at[b, 0], row_v, sem_e)
                if u > 0:
                    pltpu.make_async_copy(
                        a_loc, a_hbm.at[b, pl.ds(lo, _RQ)], sem_w).wait()

                @plsc.parallel_loop(0, _RQ, 1, unroll=2)
                def _zr(i):
                    for j in range(_NP // 16):
                        a_loc[i, pl.ds(j * 16, 16)] = zeros16

                ec.wait()

                @plsc.parallel_loop(0, _NP // 128, 1)
                def _zdeg(i):
                    for j in range(8):
                        dinv_v[pl.ds((i * 8 + j) * 16, 16)] = ones16

                @plsc.parallel_loop(0, _E // 128, 1, unroll=2)
                def _cnt(i):
                    for j in range(8):
                        cc = col_v[pl.ds((i * 8 + j) * 16, 16)]
                        plsc.addupdate_scatter(dinv_v, [cc], ones16)

                def rs(i, c):
                    for j in range(4):
                        dd = dinv_v[pl.ds((i * 4 + j) * 16, 16)]
                        bits = plsc.bitcast(dd, jnp.int32)
                        y = plsc.bitcast(
                            jnp.int32(0x5F3759DF)
                            - lax.shift_right_logical(bits, 1), jnp.float32)
                        for _ in range(3):
                            y = y * (1.5 - 0.5 * dd * y * y)
                        dinv_v[pl.ds((i * 4 + j) * 16, 16)] = y
                    return c
                lax.fori_loop(0, _NP // 64, rs, 0)
                er.wait()

                @plsc.parallel_loop(0, _E // 64, 1, unroll=2)
                def _ed(i):
                    for j in range(4):
                        sl = pl.ds((i * 4 + j) * 16, 16)
                        r = row_v[sl]
                        cc = col_v[sl]
                        m = jnp.logical_and(cc >= lo, cc < hi)
                        dr = plsc.load_gather(dinv_v, [r])
                        dc = plsc.load_gather(dinv_v, [cc])
                        cl = jnp.where(m, cc - lo, 0)
                        plsc.addupdate_scatter(a_loc, [cl, r], dr * dc, mask=m)

                for j in range(_RQ // 16):
                    lr = j * 16 + iota16
                    n = lo + lr
                    m = n < _N
                    dn = dinv_v[pl.ds(lo + j * 16, 16)]
                    plsc.addupdate_scatter(a_loc, [lr, jnp.where(m, n, 0)],
                                           dn * dn, mask=m)

                pltpu.async_copy(a_loc, a_hbm.at[b, pl.ds(lo, _RQ)], sem_w)

        pltpu.make_async_copy(a_loc, a_hbm.at[0, pl.ds(0, _RQ)], sem_w).wait()

    return k(edge_index)


def _xw1_body(x_ref, w1_ref, o_ref):
    c11 = (((1,), (1,)), ((), ()))
    r = lax.dot_general(x_ref[0], w1_ref[...], c11,
                        preferred_element_type=jnp.float32)
    o_ref[0, :_N] = r
    o_ref[0, _N:] = jnp.zeros((_NP - _N, _HID), jnp.float32)


def _xw1(xg, W1):
    return pl.pallas_call(
        _xw1_body,
        grid=(_B,),
        in_specs=[
            pl.BlockSpec((1, _N, _IN), lambda b: (b, 0, 0)),
            pl.BlockSpec((_HID, _IN), lambda b: (0, 0)),
        ],
        out_specs=pl.BlockSpec((1, _NP, _HID), lambda b: (b, 0, 0)),
        out_shape=jax.ShapeDtypeStruct((_B, _NP, _HID), jnp.float32),
    )(xg, W1)


_GB = 4


def _gcn_body(a_ref, xw_ref, b1_ref, w2_ref, b2_ref, o_ref):
    c11 = (((1,), (1,)), ((), ()))
    c10 = (((1,), (0,)), ((), ()))
    for t in range(_GB):
        av = a_ref[t]
        h1 = jnp.maximum(
            lax.dot_general(av, xw_ref[t], c10,
                            preferred_element_type=jnp.float32)
            + b1_ref[...], 0.0)
        csum = lax.dot_general(jnp.ones((1, _NP), jnp.float32), av, c10,
                               preferred_element_type=jnp.float32)
        ch1 = lax.dot_general(csum, h1, c10,
                              preferred_element_type=jnp.float32)
        o_ref[t] = lax.dot_general(ch1, w2_ref[...], c11,
                                   preferred_element_type=jnp.float32) \
            * (1.0 / _N) + b2_ref[...]


def _gcn(a, xw, b1, W2, b2):
    out = pl.pallas_call(
        _gcn_body,
        grid=(_B // _GB,),
        in_specs=[
            pl.BlockSpec((_GB, _NP, _NP), lambda b: (b, 0, 0)),
            pl.BlockSpec((_GB, _NP, _HID), lambda b: (b, 0, 0)),
            pl.BlockSpec((1, _HID), lambda b: (0, 0)),
            pl.BlockSpec((_HID, _HID), lambda b: (0, 0)),
            pl.BlockSpec((1, _HID), lambda b: (0, 0)),
        ],
        out_specs=pl.BlockSpec((_GB, 1, _HID), lambda b: (b, 0, 0)),
        out_shape=jax.ShapeDtypeStruct((_B, 1, _HID), jnp.float32),
    )(a, xw, b1.reshape(1, -1), W2, b2.reshape(1, -1))
    return out.reshape(_B, _HID)


def _head_body(x_ref, emb_ref, labr_ref, labc_ref, we_ref, be_ref,
               w1_ref, b1_ref, w2_ref, b2_ref, w3_ref, b3_ref,
               wd_ref, bd_ref, comb_ref, ximp_ref, loss_ref):
    c11 = (((1,), (1,)), ((), ()))
    f32 = jnp.float32
    x = x_ref[...]
    z = lax.dot_general(x, we_ref[...], c11, preferred_element_type=f32) \
        + be_ref[...]
    emb = emb_ref[...]
    nrm = jnp.sqrt(jnp.sum(emb * emb, axis=1, keepdims=True))
    node = emb / jnp.maximum(nrm, 1e-12)
    h = jnp.maximum(
        lax.dot_general(node, w1_ref[...], c11, preferred_element_type=f32)
        + b1_ref[...], 0.0)
    h = jnp.maximum(
        lax.dot_general(h, w2_ref[...], c11, preferred_element_type=f32)
        + b2_ref[...], 0.0)
    ximp_ref[...] = lax.dot_general(h, w3_ref[...], c11,
                                    preferred_element_type=f32) + b3_ref[...]
    comb = jnp.concatenate([z, node], axis=1)
    comb_ref[...] = comb
    logits = lax.dot_general(jnp.maximum(z, 0.0), wd_ref[...], c11,
                             preferred_element_type=f32) + bd_ref[...]
    gram = lax.dot_general(comb, comb, c11, preferred_element_type=f32)
    dist = -jnp.exp(gram * (1.0 / _TAU))
    labr = labr_ref[...]
    msk = (labr == labc_ref[...]).astype(f32)
    contrastive = jnp.sum((2.0 * msk - 1.0) * dist) / _B
    mx = jnp.max(logits, axis=1, keepdims=True)
    lse = jnp.log(jnp.sum(jnp.exp(logits - mx), axis=1, keepdims=True)) + mx
    logp = logits - lse
    oh = lax.broadcasted_iota(jnp.int32, (_B, _OUT), 1) == labr
    ce = -jnp.sum(jnp.where(oh, logp, 0.0)) / _B
    loss_ref[...] = (contrastive + ce).reshape(1, 1)


def _head(x, embs, labels, W_e, b_e, W_i1, b_i1, W_i2, b_i2, W_i3, b_i3,
          W_d, b_d):
    return pl.pallas_call(
        _head_body,
        out_shape=[
            jax.ShapeDtypeStruct((_B, 2 * _HID), jnp.float32),
            jax.ShapeDtypeStruct((_B, _IN), jnp.float32),
            jax.ShapeDtypeStruct((1, 1), jnp.float32),
        ],
    )(x, embs, labels.reshape(_B, 1), labels.reshape(1, _B),
      W_e, b_e.reshape(1, -1), W_i1, b_i1.reshape(1, -1),
      W_i2, b_i2.reshape(1, -1), W_i3, b_i3.reshape(1, -1),
      W_d, b_d.reshape(1, -1))


def kernel(x, labels, loc, subgraph_x, subgraph_edge_index, W_e, b_e,
           W_g1, b_g1, W_g2, b_g2, W_i1, b_i1, W_i2, b_i2, W_i3, b_i3,
           W_d, b_d):
    xw = _xw1(subgraph_x, W_g1)
    a = _build_adj(subgraph_edge_index)
    embs = _gcn(a, xw, b_g1, W_g2, b_g2)
    combined, x_imp, loss = _head(x, embs, labels, W_e, b_e,
                                  W_i1, b_i1, W_i2, b_i2, W_i3, b_i3,
                                  W_d, b_d)
    return combined, x_imp, loss[0, 0]
